# Initial kernel scaffold; baseline (speedup 1.0000x reference)
import functools

import jax
import jax.numpy as jnp
from jax.experimental import pallas as pl

B, DIM, N = 8, 64, 8192
T = 512
K = 4
NB = 1024


def _tc_body(src_ref, tok_ref, out_ref):
    s = src_ref[0]
    tok = tok_ref[0]
    inv_norm = jax.lax.rsqrt(jnp.sum(tok * tok, axis=0, keepdims=True))
    tn = tok * inv_norm
    scores = jax.lax.dot_general(
        s, tn, (((0,), (0,)), ((), ())),
        preferred_element_type=jnp.float32)
    iota = jax.lax.broadcasted_iota(jnp.int32, scores.shape, 1)
    keys = scores
    w = jnp.zeros_like(scores)
    for _ in range(K):
        m = jnp.max(keys, axis=1, keepdims=True)
        eq = keys == m
        first = jnp.min(jnp.where(eq, iota, T), axis=1, keepdims=True)
        sel = iota == first
        w = jnp.where(sel, 1.0 / K, w)
        keys = jnp.where(sel, -jnp.inf, keys)
    out = jax.lax.dot_general(
        tok, w, (((1,), (1,)), ((), ())),
        preferred_element_type=jnp.float32)
    out_ref[0] = out


@functools.partial(jax.jit, static_argnames=("interpret",))
def kernel(source, tokens, interpret=False):
    grid = (B, N // NB)
    return pl.pallas_call(
        _tc_body,
        grid=grid,
        in_specs=[
            pl.BlockSpec((1, DIM, NB), lambda b, nb: (b, 0, nb)),
            pl.BlockSpec((1, DIM, T), lambda b, nb: (0, 0, 0)),
        ],
        out_specs=pl.BlockSpec((1, DIM, NB), lambda b, nb: (b, 0, nb)),
        out_shape=jax.ShapeDtypeStruct((B, DIM, N), jnp.float32),
        interpret=interpret,
    )(source, tokens)


# speedup vs baseline: 26.9897x; 26.9897x over previous
import functools

import jax
import jax.numpy as jnp
from jax.experimental import pallas as pl

B, DIM, N = 8, 64, 8192
T = 512
K = 4
NB = 1024


def _tc_body(src_ref, tok_ref, out_ref):
    s = src_ref[0]
    tok = tok_ref[0]
    tn = tok / jnp.sqrt(jnp.sum(tok * tok, axis=0, keepdims=True))
    sn = s / jnp.sqrt(jnp.sum(s * s, axis=0, keepdims=True))
    scores = jax.lax.dot_general(
        sn, tn, (((0,), (0,)), ((), ())),
        preferred_element_type=jnp.float32)
    iota = jax.lax.broadcasted_iota(jnp.int32, scores.shape, 1)
    keys = scores
    w = jnp.zeros_like(scores)
    for _ in range(K):
        m = jnp.max(keys, axis=1, keepdims=True)
        eq = keys == m
        first = jnp.min(jnp.where(eq, iota, T), axis=1, keepdims=True)
        sel = iota == first
        w = jnp.where(sel, 1.0 / K, w)
        keys = jnp.where(sel, -jnp.inf, keys)
    out = jax.lax.dot_general(
        tok, w, (((1,), (1,)), ((), ())),
        preferred_element_type=jnp.float32)
    out_ref[0] = out


@functools.partial(jax.jit, static_argnames=("interpret",))
def kernel(source, tokens, interpret=False):
    grid = (B, N // NB)
    return pl.pallas_call(
        _tc_body,
        grid=grid,
        in_specs=[
            pl.BlockSpec((1, DIM, NB), lambda b, nb: (b, 0, nb)),
            pl.BlockSpec((1, DIM, T), lambda b, nb: (0, 0, 0)),
        ],
        out_specs=pl.BlockSpec((1, DIM, NB), lambda b, nb: (b, 0, nb)),
        out_shape=jax.ShapeDtypeStruct((B, DIM, N), jnp.float32),
        interpret=interpret,
    )(source, tokens)


# speedup vs baseline: 54.3527x; 2.0138x over previous
import functools

import jax
import jax.numpy as jnp
from jax.experimental import pallas as pl

B, DIM, N = 8, 64, 8192
T = 512
K = 4
NB = 1024


def _tc_body(src_ref, tok_ref, out_ref):
    s = src_ref[0]
    tok = tok_ref[0]
    tn = tok / jnp.sqrt(jnp.sum(tok * tok, axis=0, keepdims=True))
    sn = s / jnp.sqrt(jnp.sum(s * s, axis=0, keepdims=True))
    scores = jax.lax.dot_general(
        tn, sn, (((0,), (0,)), ((), ())),
        preferred_element_type=jnp.float32)
    keys = scores
    w = jnp.zeros_like(scores)
    for k in range(K):
        m = jnp.max(keys, axis=0, keepdims=True)
        lt = keys < m
        w = jnp.where(lt, w, 1.0 / K)
        if k + 1 < K:
            keys = jnp.where(lt, keys, -jnp.inf)
    out = jax.lax.dot_general(
        tok, w, (((1,), (0,)), ((), ())),
        preferred_element_type=jnp.float32)
    out_ref[0] = out


@functools.partial(jax.jit, static_argnames=("interpret",))
def kernel(source, tokens, interpret=False):
    grid = (B, N // NB)
    return pl.pallas_call(
        _tc_body,
        grid=grid,
        in_specs=[
            pl.BlockSpec((1, DIM, NB), lambda b, nb: (b, 0, nb)),
            pl.BlockSpec((1, DIM, T), lambda b, nb: (0, 0, 0)),
        ],
        out_specs=pl.BlockSpec((1, DIM, NB), lambda b, nb: (b, 0, nb)),
        out_shape=jax.ShapeDtypeStruct((B, DIM, N), jnp.float32),
        interpret=interpret,
    )(source, tokens)


# speedup vs baseline: 60.5718x; 1.1144x over previous
import functools

import jax
import jax.numpy as jnp
from jax.experimental import pallas as pl

B, DIM, N = 8, 64, 8192
T = 512
K = 4
NB = 1024


def _tc_body(src_ref, tok_ref, out_ref):
    s = src_ref[0]
    tok = tok_ref[0]
    tn = tok / jnp.sqrt(jnp.sum(tok * tok, axis=0, keepdims=True))
    sn = s / jnp.sqrt(jnp.sum(s * s, axis=0, keepdims=True))
    scores = jax.lax.dot_general(
        tn, sn, (((0,), (0,)), ((), ())),
        preferred_element_type=jnp.float32)
    m = jnp.max(scores, axis=0, keepdims=True)
    for _ in range(K - 1):
        m = jnp.max(jnp.where(scores < m, scores, -jnp.inf),
                    axis=0, keepdims=True)
    w = jnp.where(scores < m, 0.0, 1.0 / K)
    out = jax.lax.dot_general(
        tok, w, (((1,), (0,)), ((), ())),
        preferred_element_type=jnp.float32)
    out_ref[0] = out


@functools.partial(jax.jit, static_argnames=("interpret",))
def kernel(source, tokens, interpret=False):
    grid = (B, N // NB)
    return pl.pallas_call(
        _tc_body,
        grid=grid,
        in_specs=[
            pl.BlockSpec((1, DIM, NB), lambda b, nb: (b, 0, nb)),
            pl.BlockSpec((1, DIM, T), lambda b, nb: (0, 0, 0)),
        ],
        out_specs=pl.BlockSpec((1, DIM, NB), lambda b, nb: (b, 0, nb)),
        out_shape=jax.ShapeDtypeStruct((B, DIM, N), jnp.float32),
        interpret=interpret,
    )(source, tokens)


# speedup vs baseline: 73.3646x; 1.2112x over previous
import functools

import jax
import jax.numpy as jnp
from jax.experimental import pallas as pl

B, DIM, N = 8, 64, 8192
T = 512
K = 4
NB = 2048


def _tc_body(src_ref, tok_ref, out_ref):
    s = src_ref[0]
    tok = tok_ref[0]
    tn = tok / jnp.sqrt(jnp.sum(tok * tok, axis=0, keepdims=True))
    sn = s / jnp.sqrt(jnp.sum(s * s, axis=0, keepdims=True))
    scores = jax.lax.dot_general(
        tn, sn, (((0,), (0,)), ((), ())),
        preferred_element_type=jnp.float32)
    m = jnp.max(scores, axis=0, keepdims=True)
    for _ in range(K - 1):
        m = jnp.max(jnp.where(scores < m, scores, -jnp.inf),
                    axis=0, keepdims=True)
    w = jnp.where(scores < m, 0.0, 1.0 / K)
    out = jax.lax.dot_general(
        tok, w, (((1,), (0,)), ((), ())),
        preferred_element_type=jnp.float32)
    out_ref[0] = out


@functools.partial(jax.jit, static_argnames=("interpret",))
def kernel(source, tokens, interpret=False):
    grid = (B, N // NB)
    return pl.pallas_call(
        _tc_body,
        grid=grid,
        in_specs=[
            pl.BlockSpec((1, DIM, NB), lambda b, nb: (b, 0, nb)),
            pl.BlockSpec((1, DIM, T), lambda b, nb: (0, 0, 0)),
        ],
        out_specs=pl.BlockSpec((1, DIM, NB), lambda b, nb: (b, 0, nb)),
        out_shape=jax.ShapeDtypeStruct((B, DIM, N), jnp.float32),
        interpret=interpret,
    )(source, tokens)


# speedup vs baseline: 76.8888x; 1.0480x over previous
import functools

import jax
import jax.numpy as jnp
from jax.experimental import pallas as pl

B, DIM, N = 8, 64, 8192
T = 512
K = 4
NB = 4096


def _tc_body(src_ref, tok_ref, out_ref):
    s = src_ref[0]
    tok = tok_ref[0]
    tn = tok / jnp.sqrt(jnp.sum(tok * tok, axis=0, keepdims=True))
    sn = s / jnp.sqrt(jnp.sum(s * s, axis=0, keepdims=True))
    scores = jax.lax.dot_general(
        tn, sn, (((0,), (0,)), ((), ())),
        preferred_element_type=jnp.float32)
    m = jnp.max(scores, axis=0, keepdims=True)
    for _ in range(K - 1):
        m = jnp.max(jnp.where(scores < m, scores, -jnp.inf),
                    axis=0, keepdims=True)
    w = jnp.where(scores < m, 0.0, 1.0 / K)
    out = jax.lax.dot_general(
        tok, w, (((1,), (0,)), ((), ())),
        preferred_element_type=jnp.float32)
    out_ref[0] = out


@functools.partial(jax.jit, static_argnames=("interpret",))
def kernel(source, tokens, interpret=False):
    grid = (B, N // NB)
    return pl.pallas_call(
        _tc_body,
        grid=grid,
        in_specs=[
            pl.BlockSpec((1, DIM, NB), lambda b, nb: (b, 0, nb)),
            pl.BlockSpec((1, DIM, T), lambda b, nb: (0, 0, 0)),
        ],
        out_specs=pl.BlockSpec((1, DIM, NB), lambda b, nb: (b, 0, nb)),
        out_shape=jax.ShapeDtypeStruct((B, DIM, N), jnp.float32),
        interpret=interpret,
    )(source, tokens)
